# baseline (speedup 1.0000x reference)
import functools

import jax
import jax.numpy as jnp
from jax import lax
from jax.experimental import pallas as pl
from jax.experimental.pallas import tpu as pltpu
from jax.experimental.pallas import tpu_sc as plsc

VOCAB = 100000
DIM = 64
BATCH = 16384
NEG = 20

NC = 2
NS = 16
L = 16
NW = NC * NS
BPW = BATCH // NW
C = 64
NCHUNK = BPW // C
NSUB = C * NEG // 128


def _sc_body(pos_u, pos_v, neg_v, u_table, v_table, pos_out, neg_out,
             iu, iv, ineg, ru, rv, rneg, sp, sn, sem):
  wid = lax.axis_index("s") * NC + lax.axis_index("c")

  lane = lax.iota(jnp.int32, 16)
  m0 = lane == jnp.zeros((16,), jnp.int32)
  perms = [((lane + jnp.full((16,), r, jnp.int32)) &
            jnp.full((16,), 15, jnp.int32)) for r in (8, 4, 2, 1)]

  def _reduce_all(x):
    for p in perms:
      x = x + x.at[p].get(mode="promise_in_bounds")
    return x

  def chunk_body(ci, _):
    base = wid * BPW + ci * C
    pltpu.sync_copy(pos_u.at[pl.ds(base, C)], iu)
    pltpu.sync_copy(pos_v.at[pl.ds(base, C)], iv)
    pltpu.sync_copy(neg_v.at[pl.ds(base * NEG, C * NEG)], ineg)
    cps = [pltpu.async_copy(u_table.at[iu], ru, sem),
           pltpu.async_copy(v_table.at[iv], rv, sem)]
    for j in range(NSUB):
      cps.append(pltpu.async_copy(
          v_table.at[ineg.at[pl.ds(j * 128, 128)]],
          rneg.at[pl.ds(j * 128, 128)], sem))
    for cp in cps:
      cp.wait()

    def b_body(b):
      us = [ru[b, pl.ds(k * L, L)] for k in range(DIM // L)]
      nvec = jnp.full((16,), b * NEG, jnp.int32)
      bvec = jnp.full((16,), b, jnp.int32)
      prod = None
      for k in range(DIM // L):
        t = us[k] * rv[b, pl.ds(k * L, L)]
        prod = t if prod is None else prod + t
      plsc.store_scatter(sp, [bvec], _reduce_all(prod), mask=m0)
      for n in range(NEG):
        prod = None
        for k in range(DIM // L):
          t = us[k] * rneg[b * NEG + n, pl.ds(k * L, L)]
          prod = t if prod is None else prod + t
        plsc.store_scatter(sn, [nvec + n], _reduce_all(prod), mask=m0)

    plsc.parallel_loop(0, C, 1, unroll=4)(b_body)
    pltpu.sync_copy(sp, pos_out.at[pl.ds(base, C)])
    pltpu.sync_copy(sn, neg_out.at[pl.ds(base * NEG, C * NEG)])
    return 0

  lax.fori_loop(0, NCHUNK, chunk_body, 0, unroll=1)


_sc_scores = pl.kernel(
    _sc_body,
    out_type=[jax.ShapeDtypeStruct((BATCH,), jnp.float32),
              jax.ShapeDtypeStruct((BATCH * NEG,), jnp.float32)],
    mesh=plsc.VectorSubcoreMesh(core_axis_name="c", subcore_axis_name="s",
                                num_cores=NC, num_subcores=NS),
    scratch_types=[
        pltpu.VMEM((C,), jnp.int32),
        pltpu.VMEM((C,), jnp.int32),
        pltpu.VMEM((C * NEG,), jnp.int32),
        pltpu.VMEM((C, DIM), jnp.float32),
        pltpu.VMEM((C, DIM), jnp.float32),
        pltpu.VMEM((C * NEG, DIM), jnp.float32),
        pltpu.VMEM((C,), jnp.float32),
        pltpu.VMEM((C * NEG,), jnp.float32),
        pltpu.SemaphoreType.DMA,
    ],
    compiler_params=pltpu.CompilerParams(needs_layout_passes=False,
                                         use_tc_tiling_on_sc=False,
                                         disable_bounds_checks=True),
)


def _tc_body(ps_ref, ns_ref, loss_ref, ap_ref, an_ref):
  ps = ps_ref[...]
  ns = ns_ref[...]
  ap_ref[...] = 1.0 / (1.0 + jnp.exp(-ps))
  an_ref[...] = 1.0 / (1.0 + jnp.exp(ns))
  lp = jnp.minimum(ps, 0.0) - jnp.log(1.0 + jnp.exp(-jnp.abs(ps)))
  mns = -ns
  ln = jnp.minimum(mns, 0.0) - jnp.log(1.0 + jnp.exp(-jnp.abs(ns)))
  loss = -(jnp.sum(lp) + jnp.sum(ln))
  loss_ref[...] = jnp.full((1, 1), loss, jnp.float32)


_tc_post = pl.pallas_call(
    _tc_body,
    out_shape=[jax.ShapeDtypeStruct((1, 1), jnp.float32),
               jax.ShapeDtypeStruct((BATCH // 128, 128), jnp.float32),
               jax.ShapeDtypeStruct((BATCH * NEG // 128, 128), jnp.float32)],
)


def kernel(pos_u, pos_v, neg_v, u_table, v_table):
  pos_u = pos_u.astype(jnp.int32)
  pos_v = pos_v.astype(jnp.int32)
  neg_flat = neg_v.astype(jnp.int32).reshape(BATCH * NEG)
  ps, ns = _sc_scores(pos_u, pos_v, neg_flat, u_table, v_table)
  loss, ap, an = _tc_post(ps.reshape(BATCH // 128, 128),
                          ns.reshape(BATCH * NEG // 128, 128))
  return (loss.reshape(()), ap.reshape(BATCH), an.reshape(BATCH, NEG))


# speedup vs baseline: 1.0545x; 1.0545x over previous
import functools

import jax
import jax.numpy as jnp
from jax import lax
from jax.experimental import pallas as pl
from jax.experimental.pallas import tpu as pltpu
from jax.experimental.pallas import tpu_sc as plsc

VOCAB = 100000
DIM = 64
BATCH = 16384
NEG = 20

NC = 2
NS = 16
L = 16
NW = NC * NS
BPW = BATCH // NW
C = 64
NCHUNK = BPW // C
NSUB = C * NEG // 128


def _sc_body(pos_u, pos_v, neg_v, u_table, v_table, pos_out, neg_out,
             iu, iv, ineg, ru, rv, rneg, sp, sn, sem):
  wid = lax.axis_index("s") * NC + lax.axis_index("c")

  lane = lax.iota(jnp.int32, 16)
  m0 = lane == jnp.zeros((16,), jnp.int32)
  m08 = (lane == jnp.zeros((16,), jnp.int32)) | (
      lane == jnp.full((16,), 8, jnp.int32))
  perms = [((lane + jnp.full((16,), r, jnp.int32)) &
            jnp.full((16,), 15, jnp.int32)) for r in (8, 4, 2, 1)]

  def _reduce_all(x):
    for p in perms:
      x = x + x.at[p].get(mode="promise_in_bounds")
    return x

  lo8 = lane < jnp.full((16,), 8, jnp.int32)
  hperms = [((lane & jnp.full((16,), 8, jnp.int32)) |
             ((lane + jnp.full((16,), r, jnp.int32)) &
              jnp.full((16,), 7, jnp.int32))) for r in (4, 2, 1)]

  def _reduce_two(x, y):
    x = x + x.at[perms[0]].get(mode="promise_in_bounds")
    y = y + y.at[perms[0]].get(mode="promise_in_bounds")
    z = jnp.where(lo8, x, y)
    for p in hperms:
      z = z + z.at[p].get(mode="promise_in_bounds")
    return z

  def chunk_body(ci, _):
    base = wid * BPW + ci * C
    pltpu.sync_copy(pos_u.at[pl.ds(base, C)], iu)
    pltpu.sync_copy(pos_v.at[pl.ds(base, C)], iv)
    pltpu.sync_copy(neg_v.at[pl.ds(base * NEG, C * NEG)], ineg)
    cps = [pltpu.async_copy(u_table.at[iu], ru, sem),
           pltpu.async_copy(v_table.at[iv], rv, sem)]
    for j in range(NSUB):
      cps.append(pltpu.async_copy(
          v_table.at[ineg.at[pl.ds(j * 128, 128)]],
          rneg.at[pl.ds(j * 128, 128)], sem))
    for cp in cps:
      cp.wait()

    def b_body(b):
      us = [ru[b, pl.ds(k * L, L)] for k in range(DIM // L)]
      hi1 = (lane >> jnp.full((16,), 3, jnp.int32)) & jnp.full((16,), 1,
                                                              jnp.int32)
      nvecp = jnp.full((16,), b * NEG, jnp.int32) + hi1
      bvec = jnp.full((16,), b, jnp.int32)
      prod = None
      for k in range(DIM // L):
        t = us[k] * rv[b, pl.ds(k * L, L)]
        prod = t if prod is None else prod + t
      plsc.store_scatter(sp, [bvec], _reduce_all(prod), mask=m0)

      def _prod(n):
        p = None
        for k in range(DIM // L):
          t = us[k] * rneg[b * NEG + n, pl.ds(k * L, L)]
          p = t if p is None else p + t
        return p

      for n in range(0, NEG, 2):
        z = _reduce_two(_prod(n), _prod(n + 1))
        plsc.store_scatter(sn, [nvecp + n], z, mask=m08)

    plsc.parallel_loop(0, C, 1, unroll=2)(b_body)
    pltpu.sync_copy(sp, pos_out.at[pl.ds(base, C)])
    pltpu.sync_copy(sn, neg_out.at[pl.ds(base * NEG, C * NEG)])
    return 0

  lax.fori_loop(0, NCHUNK, chunk_body, 0, unroll=1)


_sc_scores = pl.kernel(
    _sc_body,
    out_type=[jax.ShapeDtypeStruct((BATCH,), jnp.float32),
              jax.ShapeDtypeStruct((BATCH * NEG,), jnp.float32)],
    mesh=plsc.VectorSubcoreMesh(core_axis_name="c", subcore_axis_name="s",
                                num_cores=NC, num_subcores=NS),
    scratch_types=[
        pltpu.VMEM((C,), jnp.int32),
        pltpu.VMEM((C,), jnp.int32),
        pltpu.VMEM((C * NEG,), jnp.int32),
        pltpu.VMEM((C, DIM), jnp.float32),
        pltpu.VMEM((C, DIM), jnp.float32),
        pltpu.VMEM((C * NEG, DIM), jnp.float32),
        pltpu.VMEM((C,), jnp.float32),
        pltpu.VMEM((C * NEG,), jnp.float32),
        pltpu.SemaphoreType.DMA,
    ],
    compiler_params=pltpu.CompilerParams(needs_layout_passes=False,
                                         use_tc_tiling_on_sc=False,
                                         disable_bounds_checks=True),
)


def _tc_body(ps_ref, ns_ref, loss_ref, ap_ref, an_ref):
  ps = ps_ref[...]
  ns = ns_ref[...]
  ap_ref[...] = 1.0 / (1.0 + jnp.exp(-ps))
  an_ref[...] = 1.0 / (1.0 + jnp.exp(ns))
  lp = jnp.minimum(ps, 0.0) - jnp.log(1.0 + jnp.exp(-jnp.abs(ps)))
  mns = -ns
  ln = jnp.minimum(mns, 0.0) - jnp.log(1.0 + jnp.exp(-jnp.abs(ns)))
  loss = -(jnp.sum(lp) + jnp.sum(ln))
  loss_ref[...] = jnp.full((1, 1), loss, jnp.float32)


_tc_post = pl.pallas_call(
    _tc_body,
    out_shape=[jax.ShapeDtypeStruct((1, 1), jnp.float32),
               jax.ShapeDtypeStruct((BATCH // 128, 128), jnp.float32),
               jax.ShapeDtypeStruct((BATCH * NEG // 128, 128), jnp.float32)],
)


def kernel(pos_u, pos_v, neg_v, u_table, v_table):
  pos_u = pos_u.astype(jnp.int32)
  pos_v = pos_v.astype(jnp.int32)
  neg_flat = neg_v.astype(jnp.int32).reshape(BATCH * NEG)
  ps, ns = _sc_scores(pos_u, pos_v, neg_flat, u_table, v_table)
  loss, ap, an = _tc_post(ps.reshape(BATCH // 128, 128),
                          ns.reshape(BATCH * NEG // 128, 128))
  return (loss.reshape(()), ap.reshape(BATCH), an.reshape(BATCH, NEG))
